# baseline (speedup 1.0000x reference)
import functools

import numpy as np
import jax
import jax.numpy as jnp
from jax import lax
from jax.experimental import pallas as pl
from jax.experimental.pallas import tpu as pltpu
from jax.experimental.pallas import tpu_sc as plsc

_B = 64
_V = 1_000_000
_BLK = 4096

_NC_TC = 190
_S_TC = _NC_TC * _BLK
_TAIL_BLK = _V // _BLK
_E_SC = _TAIL_BLK * _BLK
_W_SC = _E_SC - _S_TC
_QW = _W_SC // 4
_CHC = 2048
_NCH = _QW // _CHC
_UNR = 2
_ITI = _CHC // (16 * _UNR)
assert _QW % _CHC == 0 and _CHC % 128 == 0 and _CHC % (16 * _UNR) == 0

_M32 = 0xFFFFFFFF


def _np_threefry2x32(k0, k1, x0, x1):
    rot1 = (13, 15, 26, 6)
    rot2 = (17, 29, 16, 24)
    ks = (k0, k1, k0 ^ k1 ^ 0x1BD11BDA)

    def rnd(v0, v1, r):
        v0 = (v0 + v1) & _M32
        v1 = ((v1 << r) | (v1 >> (32 - r))) & _M32
        return v0, v0 ^ v1

    x0 = (x0 + ks[0]) & _M32
    x1 = (x1 + ks[1]) & _M32
    for i, (ka, kb) in enumerate(
        ((ks[1], ks[2]), (ks[2], ks[0]), (ks[0], ks[1]),
         (ks[1], ks[2]), (ks[2], ks[0]))):
        rots = rot1 if i % 2 == 0 else rot2
        for r in rots:
            x0, x1 = rnd(x0, x1, r)
        x0 = (x0 + ka) & _M32
        x1 = (x1 + kb + i + 1) & _M32
    return x0, x1


_SK0, _SK1 = _np_threefry2x32(0, 42, 0, 0)
_SK2 = _SK0 ^ _SK1 ^ 0x1BD11BDA

_TINY = np.float32(np.finfo(np.float32).tiny)


def _rotl(x, r):
    return (x << np.uint32(r)) | (x >> np.uint32(32 - r))


def _threefry_bits(j):
    rot1 = (13, 15, 26, 6)
    rot2 = (17, 29, 16, 24)
    ks = (np.uint32(_SK0), np.uint32(_SK1), np.uint32(_SK2))
    x0 = jnp.full(j.shape, ks[0], jnp.uint32)
    x1 = j + ks[1]
    for i, (ka, kb) in enumerate(
        ((ks[1], ks[2]), (ks[2], ks[0]), (ks[0], ks[1]),
         (ks[1], ks[2]), (ks[2], ks[0]))):
        rots = rot1 if i % 2 == 0 else rot2
        for r in rots:
            x0 = x0 + x1
            x1 = x0 ^ _rotl(x1, r)
        x0 = x0 + ka
        x1 = x1 + np.uint32((int(kb) + i + 1) & _M32)
    return x0 ^ x1


def _bits_to_u(bits):
    fb = (bits >> np.uint32(9)) | np.uint32(0x3F800000)
    f = lax.bitcast_convert_type(fb, jnp.float32) - np.float32(1.0)
    return jnp.maximum(f, _TINY)


def _tc_kernel(w_ref, val_ref, idx_ref, sv_ref, si_ref):
    c = pl.program_id(0)

    @pl.when(c == 0)
    def _init():
        sv_ref[...] = jnp.full((_B, _BLK), -1.0, jnp.float32)
        si_ref[...] = jnp.zeros((_B, _BLK), jnp.int32)

    base = jnp.where(c == _NC_TC, np.int32(_TAIL_BLK * _BLK),
                     c * np.int32(_BLK))
    cols = lax.broadcasted_iota(jnp.int32, (_B, _BLK), 1) + base
    rows = lax.broadcasted_iota(jnp.int32, (_B, _BLK), 0)
    j = (rows * np.int32(_V) + cols).astype(jnp.uint32)

    u = _bits_to_u(_threefry_bits(j))
    e = -jnp.log(u)
    wp = jnp.maximum(w_ref[...], np.float32(1e-30))
    val = wp / e

    def _combine(v):
        sv = sv_ref[...]
        take = v > sv
        sv_ref[...] = jnp.where(take, v, sv)
        si_ref[...] = jnp.where(take, cols, si_ref[...])

    @pl.when(c != _NC_TC)
    def _bulk():
        _combine(val)

    @pl.when(c == _NC_TC)
    def _tail():
        _combine(jnp.where(cols < _V, val, np.float32(-1.0)))

    @pl.when(c == _NC_TC)
    def _finish():
        v = sv_ref[...]
        idx = si_ref[...]
        mv = jnp.max(v, axis=1, keepdims=True)
        mi = jnp.min(jnp.where(v == mv, idx, np.int32(0x7FFFFFFF)),
                     axis=1, keepdims=True)
        val_ref[...] = mv
        idx_ref[...] = mi


def _tc_partial(input):
    return pl.pallas_call(
        _tc_kernel,
        grid=(_NC_TC + 1,),
        in_specs=[pl.BlockSpec(
            (_B, _BLK),
            lambda c: (0, jnp.where(c == _NC_TC, _TAIL_BLK, c)))],
        out_specs=[pl.BlockSpec((_B, 1), lambda c: (0, 0)),
                   pl.BlockSpec((_B, 1), lambda c: (0, 0))],
        out_shape=[jax.ShapeDtypeStruct((_B, 1), jnp.float32),
                   jax.ShapeDtypeStruct((_B, 1), jnp.int32)],
        scratch_shapes=[
            pltpu.VMEM((_B, _BLK), jnp.float32),
            pltpu.VMEM((_B, _BLK), jnp.int32),
        ],
        compiler_params=pltpu.CompilerParams(
            dimension_semantics=("arbitrary",),
        ),
    )(input)


_LN2 = np.float32(0.6931471805599453)
_SQRT2 = np.float32(1.4142135623730951)


def _sc_log(u):
    ib = lax.bitcast_convert_type(u, jnp.int32)
    ex = (ib >> np.int32(23)) - np.int32(127)
    m = lax.bitcast_convert_type(
        (ib & np.int32(0x007FFFFF)) | np.int32(0x3F800000), jnp.float32)
    big = m > _SQRT2
    ex = jnp.where(big, ex + np.int32(1), ex)
    m = jnp.where(big, m * np.float32(0.5), m)
    z = (m - np.float32(1.0)) / (m + np.float32(1.0))
    y = z * z
    p = np.float32(1.0 / 9.0)
    p = p * y + np.float32(1.0 / 7.0)
    p = p * y + np.float32(1.0 / 5.0)
    p = p * y + np.float32(1.0 / 3.0)
    p = p * y + np.float32(1.0)
    return ex.astype(jnp.float32) * _LN2 + p * (z + z)


def _sc_kernel_body(in_hbm, val_hbm, idx_hbm, buf, bv_ref, bi_ref):
    cid = lax.axis_index("c")
    sid = lax.axis_index("s")
    wid = sid * 2 + cid
    band = wid // 4
    q = wid % 4
    row0 = pl.multiple_of(band * 8, 8)
    colq = pl.multiple_of(np.int32(_S_TC) + q * np.int32(_QW), 128)

    bv_ref[...] = jnp.full((8, 16), -1.0, jnp.float32)
    bi_ref[...] = jnp.zeros((8, 16), jnp.int32)

    def chunk(k, _):
        col0 = pl.multiple_of(colq + k * np.int32(_CHC), 128)
        pltpu.sync_copy(in_hbm.at[pl.ds(row0, 8), pl.ds(col0, _CHC)], buf)
        for rr in range(8):
            j_row = (row0 + rr) * np.int32(_V)

            def inner(i, _, rr=rr, j_row=j_row, col0=col0):
                lane = lax.iota(jnp.int32, 16)
                bv2 = bv_ref[rr]
                bi2 = bi_ref[rr]
                for s in range(_UNR):
                    off = i * np.int32(16 * _UNR) + np.int32(16 * s)
                    w16 = buf[rr, pl.ds(off, 16)]
                    col = col0 + off + lane
                    jj = (j_row + col).astype(jnp.uint32)
                    u = _bits_to_u(_threefry_bits(jj))
                    e = -_sc_log(u)
                    wp = jnp.maximum(w16, np.float32(1e-30))
                    v = wp / e
                    take = v > bv2
                    bv2 = jnp.where(take, v, bv2)
                    bi2 = jnp.where(take, col, bi2)
                bv_ref[rr] = bv2
                bi_ref[rr] = bi2
                return 0

            lax.fori_loop(0, _ITI, inner, 0)
        return 0

    lax.fori_loop(0, _NCH, chunk, 0)

    for rr in range(8):
        o = pl.multiple_of(((row0 + rr) * 4 + q) * 16, 8)
        pltpu.sync_copy(bv_ref.at[rr], val_hbm.at[pl.ds(o, 16)])
        pltpu.sync_copy(bi_ref.at[rr], idx_hbm.at[pl.ds(o, 16)])


@functools.cache
def _sc_partial():
    return pl.kernel(
        _sc_kernel_body,
        out_type=(jax.ShapeDtypeStruct((_B * 64,), jnp.float32),
                  jax.ShapeDtypeStruct((_B * 64,), jnp.int32)),
        mesh=plsc.VectorSubcoreMesh(core_axis_name="c",
                                    subcore_axis_name="s"),
        scratch_types=[
            pltpu.VMEM((8, _CHC), jnp.float32),
            pltpu.VMEM((8, 16), jnp.float32),
            pltpu.VMEM((8, 16), jnp.int32),
        ],
    )


def kernel(input):
    tc_val, tc_idx = _tc_partial(input)
    sc_val, sc_idx = _sc_partial()(input)
    sc_val = sc_val.reshape(_B, 64)
    sc_idx = sc_idx.reshape(_B, 64)
    mv = jnp.max(sc_val, axis=1, keepdims=True)
    mi = jnp.min(jnp.where(sc_val == mv, sc_idx, np.int32(0x7FFFFFFF)),
                 axis=1, keepdims=True)
    return jnp.where(mv > tc_val, mi, tc_idx)


# speedup vs baseline: 1.6199x; 1.6199x over previous
import functools

import numpy as np
import jax
import jax.numpy as jnp
from jax import lax
from jax.experimental import pallas as pl
from jax.experimental.pallas import tpu as pltpu
from jax.experimental.pallas import tpu_sc as plsc

_B = 64
_V = 1_000_000
_BLK = 4096

_NC_TC = 190
_S_TC = _NC_TC * _BLK
_TAIL_BLK = _V // _BLK
_E_SC = _TAIL_BLK * _BLK
_W_SC = _E_SC - _S_TC
_QW = _W_SC // 4
_CHC = 2048
_NCH = _QW // _CHC
_UNR = 2
_ITI = _CHC // (16 * _UNR)
assert _QW % _CHC == 0 and _CHC % 128 == 0 and _CHC % (16 * _UNR) == 0

_M32 = 0xFFFFFFFF


def _np_threefry2x32(k0, k1, x0, x1):
    rot1 = (13, 15, 26, 6)
    rot2 = (17, 29, 16, 24)
    ks = (k0, k1, k0 ^ k1 ^ 0x1BD11BDA)

    def rnd(v0, v1, r):
        v0 = (v0 + v1) & _M32
        v1 = ((v1 << r) | (v1 >> (32 - r))) & _M32
        return v0, v0 ^ v1

    x0 = (x0 + ks[0]) & _M32
    x1 = (x1 + ks[1]) & _M32
    for i, (ka, kb) in enumerate(
        ((ks[1], ks[2]), (ks[2], ks[0]), (ks[0], ks[1]),
         (ks[1], ks[2]), (ks[2], ks[0]))):
        rots = rot1 if i % 2 == 0 else rot2
        for r in rots:
            x0, x1 = rnd(x0, x1, r)
        x0 = (x0 + ka) & _M32
        x1 = (x1 + kb + i + 1) & _M32
    return x0, x1


_SK0, _SK1 = _np_threefry2x32(0, 42, 0, 0)
_SK2 = _SK0 ^ _SK1 ^ 0x1BD11BDA

_TINY = np.float32(np.finfo(np.float32).tiny)


def _rotl(x, r):
    return (x << np.uint32(r)) | (x >> np.uint32(32 - r))


def _threefry_bits(j):
    rot1 = (13, 15, 26, 6)
    rot2 = (17, 29, 16, 24)
    ks = (np.uint32(_SK0), np.uint32(_SK1), np.uint32(_SK2))
    x0 = jnp.full(j.shape, ks[0], jnp.uint32)
    x1 = j + ks[1]
    for i, (ka, kb) in enumerate(
        ((ks[1], ks[2]), (ks[2], ks[0]), (ks[0], ks[1]),
         (ks[1], ks[2]), (ks[2], ks[0]))):
        rots = rot1 if i % 2 == 0 else rot2
        for r in rots:
            x0 = x0 + x1
            x1 = x0 ^ _rotl(x1, r)
        x0 = x0 + ka
        x1 = x1 + np.uint32((int(kb) + i + 1) & _M32)
    return x0 ^ x1


def _bits_to_u(bits):
    fb = (bits >> np.uint32(9)) | np.uint32(0x3F800000)
    f = lax.bitcast_convert_type(fb, jnp.float32) - np.float32(1.0)
    return jnp.maximum(f, _TINY)


def _tc_kernel(w_ref, val_ref, idx_ref, sv_ref, si_ref):
    c = pl.program_id(0)

    @pl.when(c == 0)
    def _init():
        sv_ref[...] = jnp.full((_B, _BLK), -1.0, jnp.float32)
        si_ref[...] = jnp.zeros((_B, _BLK), jnp.int32)

    base = jnp.where(c == _NC_TC, np.int32(_TAIL_BLK * _BLK),
                     c * np.int32(_BLK))
    cols = lax.broadcasted_iota(jnp.int32, (_B, _BLK), 1) + base
    rows = lax.broadcasted_iota(jnp.int32, (_B, _BLK), 0)
    j = (rows * np.int32(_V) + cols).astype(jnp.uint32)

    u = _bits_to_u(_threefry_bits(j))
    e = -jnp.log(u)
    wp = jnp.maximum(w_ref[...], np.float32(1e-30))
    val = wp / e
    val = jnp.where(cols < _V, val, np.float32(-1.0))

    sv = sv_ref[...]
    take = val > sv
    sv_ref[...] = jnp.where(take, val, sv)
    si_ref[...] = jnp.where(take, cols, si_ref[...])

    @pl.when(c == _NC_TC)
    def _finish():
        v = sv_ref[...]
        idx = si_ref[...]
        mv = jnp.max(v, axis=1, keepdims=True)
        mi = jnp.min(jnp.where(v == mv, idx, np.int32(0x7FFFFFFF)),
                     axis=1, keepdims=True)
        val_ref[...] = mv
        idx_ref[...] = mi


def _tc_partial(input):
    return pl.pallas_call(
        _tc_kernel,
        grid=(_NC_TC + 1,),
        in_specs=[pl.BlockSpec(
            (_B, _BLK),
            lambda c: (0, jnp.where(c == _NC_TC, _TAIL_BLK, c)))],
        out_specs=[pl.BlockSpec((_B, 1), lambda c: (0, 0)),
                   pl.BlockSpec((_B, 1), lambda c: (0, 0))],
        out_shape=[jax.ShapeDtypeStruct((_B, 1), jnp.float32),
                   jax.ShapeDtypeStruct((_B, 1), jnp.int32)],
        scratch_shapes=[
            pltpu.VMEM((_B, _BLK), jnp.float32),
            pltpu.VMEM((_B, _BLK), jnp.int32),
        ],
        compiler_params=pltpu.CompilerParams(
            dimension_semantics=("arbitrary",),
        ),
    )(input)


_LN2 = np.float32(0.6931471805599453)
_SQRT2 = np.float32(1.4142135623730951)


def _sc_log(u):
    ib = lax.bitcast_convert_type(u, jnp.int32)
    ex = (ib >> np.int32(23)) - np.int32(127)
    m = lax.bitcast_convert_type(
        (ib & np.int32(0x007FFFFF)) | np.int32(0x3F800000), jnp.float32)
    big = m > _SQRT2
    ex = jnp.where(big, ex + np.int32(1), ex)
    m = jnp.where(big, m * np.float32(0.5), m)
    z = (m - np.float32(1.0)) / (m + np.float32(1.0))
    y = z * z
    p = np.float32(1.0 / 9.0)
    p = p * y + np.float32(1.0 / 7.0)
    p = p * y + np.float32(1.0 / 5.0)
    p = p * y + np.float32(1.0 / 3.0)
    p = p * y + np.float32(1.0)
    return ex.astype(jnp.float32) * _LN2 + p * (z + z)


def _sc_kernel_body(in_hbm, val_hbm, idx_hbm, buf, bv_ref, bi_ref):
    cid = lax.axis_index("c")
    sid = lax.axis_index("s")
    wid = sid * 2 + cid
    band = wid // 4
    q = wid % 4
    row0 = pl.multiple_of(band * 8, 8)
    colq = pl.multiple_of(np.int32(_S_TC) + q * np.int32(_QW), 128)

    bv_ref[...] = jnp.full((8, 16), -1.0, jnp.float32)
    bi_ref[...] = jnp.zeros((8, 16), jnp.int32)

    def chunk(k, _):
        col0 = pl.multiple_of(colq + k * np.int32(_CHC), 128)
        pltpu.sync_copy(in_hbm.at[pl.ds(row0, 8), pl.ds(col0, _CHC)], buf)
        for rr in range(8):
            j_row = (row0 + rr) * np.int32(_V)

            def inner(i, _, rr=rr, j_row=j_row, col0=col0):
                lane = lax.iota(jnp.int32, 16)
                bv2 = bv_ref[rr]
                bi2 = bi_ref[rr]
                for s in range(_UNR):
                    off = i * np.int32(16 * _UNR) + np.int32(16 * s)
                    w16 = buf[rr, pl.ds(off, 16)]
                    col = col0 + off + lane
                    jj = (j_row + col).astype(jnp.uint32)
                    u = _bits_to_u(_threefry_bits(jj))
                    e = -_sc_log(u)
                    wp = jnp.maximum(w16, np.float32(1e-30))
                    v = wp / e
                    take = v > bv2
                    bv2 = jnp.where(take, v, bv2)
                    bi2 = jnp.where(take, col, bi2)
                bv_ref[rr] = bv2
                bi_ref[rr] = bi2
                return 0

            lax.fori_loop(0, _ITI, inner, 0)
        return 0

    lax.fori_loop(0, _NCH, chunk, 0)

    for rr in range(8):
        o = pl.multiple_of(((row0 + rr) * 4 + q) * 16, 8)
        pltpu.sync_copy(bv_ref.at[rr], val_hbm.at[pl.ds(o, 16)])
        pltpu.sync_copy(bi_ref.at[rr], idx_hbm.at[pl.ds(o, 16)])


@functools.cache
def _sc_partial():
    return pl.kernel(
        _sc_kernel_body,
        out_type=(jax.ShapeDtypeStruct((_B * 64,), jnp.float32),
                  jax.ShapeDtypeStruct((_B * 64,), jnp.int32)),
        mesh=plsc.VectorSubcoreMesh(core_axis_name="c",
                                    subcore_axis_name="s"),
        scratch_types=[
            pltpu.VMEM((8, _CHC), jnp.float32),
            pltpu.VMEM((8, 16), jnp.float32),
            pltpu.VMEM((8, 16), jnp.int32),
        ],
    )


def kernel(input):
    tc_val, tc_idx = _tc_partial(input)
    sc_val, sc_idx = _sc_partial()(input)
    sc_val = sc_val.reshape(_B, 64)
    sc_idx = sc_idx.reshape(_B, 64)
    mv = jnp.max(sc_val, axis=1, keepdims=True)
    mi = jnp.min(jnp.where(sc_val == mv, sc_idx, np.int32(0x7FFFFFFF)),
                 axis=1, keepdims=True)
    return jnp.where(mv > tc_val, mi, tc_idx)


# speedup vs baseline: 1.6492x; 1.0181x over previous
import functools

import numpy as np
import jax
import jax.numpy as jnp
from jax import lax
from jax.experimental import pallas as pl
from jax.experimental.pallas import tpu as pltpu
from jax.experimental.pallas import tpu_sc as plsc

_B = 64
_V = 1_000_000
_BLK = 4096

_NC_TC = 192
_S_TC = _NC_TC * _BLK
_TAIL_BLK = _V // _BLK
_E_SC = _TAIL_BLK * _BLK
_W_SC = _E_SC - _S_TC
_QW = _W_SC // 4
_CHC = 4096
_NCH = _QW // _CHC
_UNR = 2
_ITI = _CHC // (16 * _UNR)
assert _QW % _CHC == 0 and _CHC % 128 == 0 and _CHC % (16 * _UNR) == 0

_M32 = 0xFFFFFFFF


def _np_threefry2x32(k0, k1, x0, x1):
    rot1 = (13, 15, 26, 6)
    rot2 = (17, 29, 16, 24)
    ks = (k0, k1, k0 ^ k1 ^ 0x1BD11BDA)

    def rnd(v0, v1, r):
        v0 = (v0 + v1) & _M32
        v1 = ((v1 << r) | (v1 >> (32 - r))) & _M32
        return v0, v0 ^ v1

    x0 = (x0 + ks[0]) & _M32
    x1 = (x1 + ks[1]) & _M32
    for i, (ka, kb) in enumerate(
        ((ks[1], ks[2]), (ks[2], ks[0]), (ks[0], ks[1]),
         (ks[1], ks[2]), (ks[2], ks[0]))):
        rots = rot1 if i % 2 == 0 else rot2
        for r in rots:
            x0, x1 = rnd(x0, x1, r)
        x0 = (x0 + ka) & _M32
        x1 = (x1 + kb + i + 1) & _M32
    return x0, x1


_SK0, _SK1 = _np_threefry2x32(0, 42, 0, 0)
_SK2 = _SK0 ^ _SK1 ^ 0x1BD11BDA

_TINY = np.float32(np.finfo(np.float32).tiny)


def _rotl(x, r):
    return (x << np.uint32(r)) | (x >> np.uint32(32 - r))


def _threefry_bits(j):
    rot1 = (13, 15, 26, 6)
    rot2 = (17, 29, 16, 24)
    ks = (np.uint32(_SK0), np.uint32(_SK1), np.uint32(_SK2))
    x0 = jnp.full(j.shape, ks[0], jnp.uint32)
    x1 = j + ks[1]
    for i, (ka, kb) in enumerate(
        ((ks[1], ks[2]), (ks[2], ks[0]), (ks[0], ks[1]),
         (ks[1], ks[2]), (ks[2], ks[0]))):
        rots = rot1 if i % 2 == 0 else rot2
        for r in rots:
            x0 = x0 + x1
            x1 = x0 ^ _rotl(x1, r)
        x0 = x0 + ka
        x1 = x1 + np.uint32((int(kb) + i + 1) & _M32)
    return x0 ^ x1


def _bits_to_u(bits):
    fb = (bits >> np.uint32(9)) | np.uint32(0x3F800000)
    f = lax.bitcast_convert_type(fb, jnp.float32) - np.float32(1.0)
    return jnp.maximum(f, _TINY)


def _tc_kernel(w_ref, val_ref, idx_ref, sv_ref, si_ref):
    c = pl.program_id(0)

    @pl.when(c == 0)
    def _init():
        sv_ref[...] = jnp.full((_B, _BLK), -1.0, jnp.float32)
        si_ref[...] = jnp.zeros((_B, _BLK), jnp.int32)

    base = jnp.where(c == _NC_TC, np.int32(_TAIL_BLK * _BLK),
                     c * np.int32(_BLK))
    cols = lax.broadcasted_iota(jnp.int32, (_B, _BLK), 1) + base
    rows = lax.broadcasted_iota(jnp.int32, (_B, _BLK), 0)
    j = (rows * np.int32(_V) + cols).astype(jnp.uint32)

    u = _bits_to_u(_threefry_bits(j))
    e = -jnp.log(u)
    wp = jnp.maximum(w_ref[...], np.float32(1e-30))
    val = wp / e
    val = jnp.where(cols < _V, val, np.float32(-1.0))

    sv = sv_ref[...]
    take = val > sv
    sv_ref[...] = jnp.where(take, val, sv)
    si_ref[...] = jnp.where(take, cols, si_ref[...])

    @pl.when(c == _NC_TC)
    def _finish():
        v = sv_ref[...]
        idx = si_ref[...]
        mv = jnp.max(v, axis=1, keepdims=True)
        mi = jnp.min(jnp.where(v == mv, idx, np.int32(0x7FFFFFFF)),
                     axis=1, keepdims=True)
        val_ref[...] = mv
        idx_ref[...] = mi


def _tc_partial(input):
    return pl.pallas_call(
        _tc_kernel,
        grid=(_NC_TC + 1,),
        in_specs=[pl.BlockSpec(
            (_B, _BLK),
            lambda c: (0, jnp.where(c == _NC_TC, _TAIL_BLK, c)))],
        out_specs=[pl.BlockSpec((_B, 1), lambda c: (0, 0)),
                   pl.BlockSpec((_B, 1), lambda c: (0, 0))],
        out_shape=[jax.ShapeDtypeStruct((_B, 1), jnp.float32),
                   jax.ShapeDtypeStruct((_B, 1), jnp.int32)],
        scratch_shapes=[
            pltpu.VMEM((_B, _BLK), jnp.float32),
            pltpu.VMEM((_B, _BLK), jnp.int32),
        ],
        compiler_params=pltpu.CompilerParams(
            dimension_semantics=("arbitrary",),
        ),
    )(input)


_LN2 = np.float32(0.6931471805599453)
_SQRT2 = np.float32(1.4142135623730951)


def _sc_log(u):
    ib = lax.bitcast_convert_type(u, jnp.int32)
    ex = (ib >> np.int32(23)) - np.int32(127)
    m = lax.bitcast_convert_type(
        (ib & np.int32(0x007FFFFF)) | np.int32(0x3F800000), jnp.float32)
    big = m > _SQRT2
    ex = jnp.where(big, ex + np.int32(1), ex)
    m = jnp.where(big, m * np.float32(0.5), m)
    z = (m - np.float32(1.0)) / (m + np.float32(1.0))
    y = z * z
    p = np.float32(1.0 / 9.0)
    p = p * y + np.float32(1.0 / 7.0)
    p = p * y + np.float32(1.0 / 5.0)
    p = p * y + np.float32(1.0 / 3.0)
    p = p * y + np.float32(1.0)
    return ex.astype(jnp.float32) * _LN2 + p * (z + z)


def _sc_kernel_body(in_hbm, val_hbm, idx_hbm, buf, bv_ref, bi_ref):
    cid = lax.axis_index("c")
    sid = lax.axis_index("s")
    wid = sid * 2 + cid
    band = wid // 4
    q = wid % 4
    row0 = pl.multiple_of(band * 8, 8)
    colq = pl.multiple_of(np.int32(_S_TC) + q * np.int32(_QW), 128)

    bv_ref[...] = jnp.full((8, 16), -1.0, jnp.float32)
    bi_ref[...] = jnp.zeros((8, 16), jnp.int32)

    def chunk(k, _):
        col0 = pl.multiple_of(colq + k * np.int32(_CHC), 128)
        pltpu.sync_copy(in_hbm.at[pl.ds(row0, 8), pl.ds(col0, _CHC)], buf)
        for rr in range(8):
            j_row = (row0 + rr) * np.int32(_V)

            def inner(i, _, rr=rr, j_row=j_row, col0=col0):
                lane = lax.iota(jnp.int32, 16)
                bv2 = bv_ref[rr]
                bi2 = bi_ref[rr]
                for s in range(_UNR):
                    off = i * np.int32(16 * _UNR) + np.int32(16 * s)
                    w16 = buf[rr, pl.ds(off, 16)]
                    col = col0 + off + lane
                    jj = (j_row + col).astype(jnp.uint32)
                    u = _bits_to_u(_threefry_bits(jj))
                    e = -_sc_log(u)
                    wp = jnp.maximum(w16, np.float32(1e-30))
                    v = wp / e
                    take = v > bv2
                    bv2 = jnp.where(take, v, bv2)
                    bi2 = jnp.where(take, col, bi2)
                bv_ref[rr] = bv2
                bi_ref[rr] = bi2
                return 0

            lax.fori_loop(0, _ITI, inner, 0)
        return 0

    lax.fori_loop(0, _NCH, chunk, 0)

    for rr in range(8):
        o = pl.multiple_of(((row0 + rr) * 4 + q) * 16, 8)
        pltpu.sync_copy(bv_ref.at[rr], val_hbm.at[pl.ds(o, 16)])
        pltpu.sync_copy(bi_ref.at[rr], idx_hbm.at[pl.ds(o, 16)])


@functools.cache
def _sc_partial():
    return pl.kernel(
        _sc_kernel_body,
        out_type=(jax.ShapeDtypeStruct((_B * 64,), jnp.float32),
                  jax.ShapeDtypeStruct((_B * 64,), jnp.int32)),
        mesh=plsc.VectorSubcoreMesh(core_axis_name="c",
                                    subcore_axis_name="s"),
        scratch_types=[
            pltpu.VMEM((8, _CHC), jnp.float32),
            pltpu.VMEM((8, 16), jnp.float32),
            pltpu.VMEM((8, 16), jnp.int32),
        ],
    )


def kernel(input):
    tc_val, tc_idx = _tc_partial(input)
    sc_val, sc_idx = _sc_partial()(input)
    sc_val = sc_val.reshape(_B, 64)
    sc_idx = sc_idx.reshape(_B, 64)
    mv = jnp.max(sc_val, axis=1, keepdims=True)
    mi = jnp.min(jnp.where(sc_val == mv, sc_idx, np.int32(0x7FFFFFFF)),
                 axis=1, keepdims=True)
    return jnp.where(mv > tc_val, mi, tc_idx)
